# Initial kernel scaffold; baseline (speedup 1.0000x reference)
import jax
import jax.numpy as jnp
from jax.experimental import pallas as pl


def kernel(x, edge_index, edge_attr, batch, params):
    raise NotImplementedError("write your pallas kernel here")


# speedup vs baseline: 1.4753x; 1.4753x over previous
import jax
import jax.numpy as jnp
import numpy as np
from jax.experimental import pallas as pl

NODE_DIM = 128
EDGE_DIM = 16
HIDDEN = 256
HEADS = 8
DH = HIDDEN // HEADS
N_GRAPHS = 64
NPG = 64
EPG = 256
NEG_INF = -1e9


def _ln(h, g, b):
    m = jnp.mean(h, axis=-1, keepdims=True)
    d = h - m
    v = jnp.mean(d * d, axis=-1, keepdims=True)
    return d * jax.lax.rsqrt(v + 1e-5) * g + b


def _mha(q_in, kv_in, wq, bq, wk, bk, wv, bv, wo, bo, adj):
    q = jnp.dot(q_in, wq, preferred_element_type=jnp.float32) + bq
    k = jnp.dot(kv_in, wk, preferred_element_type=jnp.float32) + bk
    v = jnp.dot(kv_in, wv, preferred_element_type=jnp.float32) + bv
    scale = 1.0 / np.sqrt(DH)
    outs = []
    for hh in range(HEADS):
        qh = q[:, hh * DH:(hh + 1) * DH]
        kh = k[:, hh * DH:(hh + 1) * DH]
        vh = v[:, hh * DH:(hh + 1) * DH]
        s = jax.lax.dot_general(qh, kh, (((1,), (1,)), ((), ())),
                                preferred_element_type=jnp.float32) * scale
        if adj is not None:
            s = jnp.where(adj, s, NEG_INF)
        m = jnp.max(s, axis=-1, keepdims=True)
        e = jnp.exp(s - m)
        a = e / jnp.sum(e, axis=-1, keepdims=True)
        outs.append(jnp.dot(a, vh, preferred_element_type=jnp.float32))
    o = jnp.concatenate(outs, axis=-1)
    return jnp.dot(o, wo, preferred_element_type=jnp.float32) + bo


def _ff(h1, f1w, f1b, f2w, f2b):
    t = jnp.maximum(jnp.dot(h1, f1w, preferred_element_type=jnp.float32) + f1b, 0.0)
    return jnp.dot(t, f2w, preferred_element_type=jnp.float32) + f2b


def _body(xg_ref, sc_ref, sr_ref, dc_ref, dr_ref, ea_ref,
          w1a_ref, w1b_ref, w1c_ref, b1_ref, w2_ref, b2_ref,
          wq_ref, wk_ref, wv_ref, wo_ref, f1w_ref, f2w_ref,
          bq_ref, bk_ref, bv_ref, bo_ref, f1b_ref, f2b_ref,
          l1g_ref, l1b_ref, l2g_ref, l2b_ref,
          seed_ref, ow1_ref, ob1_ref, ow2_ref, ob2_ref,
          out_ref):
    xg = xg_ref[0]
    sc = sc_ref[0]
    sr = sr_ref[0]
    dc = dc_ref[0]
    dr = dr_ref[0]
    ea = ea_ref[0]

    adj = (sc == sr) | (sc == dr) | (dc == sr) | (dc == dr)

    iota = jax.lax.broadcasted_iota(jnp.int32, (EPG, NPG), 1)
    ohs = (sc == iota).astype(jnp.float32)
    ohd = (dc == iota).astype(jnp.float32)
    a_proj = jnp.dot(xg, w1a_ref[...], preferred_element_type=jnp.float32)
    b_proj = jnp.dot(xg, w1b_ref[...], preferred_element_type=jnp.float32)
    pre = (jnp.dot(ohs, a_proj, preferred_element_type=jnp.float32)
           + jnp.dot(ohd, b_proj, preferred_element_type=jnp.float32)
           + jnp.dot(ea, w1c_ref[...], preferred_element_type=jnp.float32)
           + b1_ref[...])
    h = jnp.dot(jnp.maximum(pre, 0.0), w2_ref[...],
                preferred_element_type=jnp.float32) + b2_ref[...]

    for i in range(3):
        mask = adj if i < 2 else None
        att = _mha(h, h, wq_ref[i], bq_ref[i], wk_ref[i], bk_ref[i],
                   wv_ref[i], bv_ref[i], wo_ref[i], bo_ref[i], mask)
        h1 = _ln(h + att, l1g_ref[i], l1b_ref[i])
        f = _ff(h1, f1w_ref[i], f1b_ref[i], f2w_ref[i], f2b_ref[i])
        h = _ln(h1 + f, l2g_ref[i], l2b_ref[i])

    s_seed = seed_ref[...]
    att = _mha(s_seed, h, wq_ref[3], bq_ref[3], wk_ref[3], bk_ref[3],
               wv_ref[3], bv_ref[3], wo_ref[3], bo_ref[3], None)
    h1 = _ln(s_seed + att, l1g_ref[3], l1b_ref[3])
    f = _ff(h1, f1w_ref[3], f1b_ref[3], f2w_ref[3], f2b_ref[3])
    pooled = _ln(h1 + f, l2g_ref[3], l2b_ref[3])

    t = jnp.maximum(jnp.dot(pooled, ow1_ref[...],
                            preferred_element_type=jnp.float32) + ob1_ref[...], 0.0)
    logit = jnp.sum(t * ow2_ref[...], axis=-1, keepdims=True)
    out_ref[0] = jnp.broadcast_to(logit, (1, 128)) + ob2_ref[...]


def _im3(g):
    return (g, 0, 0)


def _c2(g):
    return (0, 0)


def _c3(g):
    return (0, 0, 0)


def kernel(x, edge_index, edge_attr, batch, params):
    src = edge_index[0].astype(jnp.int32)
    dst = edge_index[1].astype(jnp.int32)
    base = (jnp.arange(N_GRAPHS, dtype=jnp.int32) * NPG)[:, None]
    src_l = src.reshape(N_GRAPHS, EPG) - base
    dst_l = dst.reshape(N_GRAPHS, EPG) - base
    sc = src_l.reshape(N_GRAPHS, EPG, 1)
    sr = src_l.reshape(N_GRAPHS, 1, EPG)
    dc = dst_l.reshape(N_GRAPHS, EPG, 1)
    dr = dst_l.reshape(N_GRAPHS, 1, EPG)
    x3 = x.reshape(N_GRAPHS, NPG, NODE_DIM)
    ea = jnp.pad(edge_attr, ((0, 0), (0, 128 - EDGE_DIM))).reshape(N_GRAPHS, EPG, 128)

    p = params
    w1 = p['in_mlp']['W1']
    w1a = w1[:NODE_DIM]
    w1b = w1[NODE_DIM:2 * NODE_DIM]
    w1c = jnp.pad(w1[2 * NODE_DIM:], ((0, 128 - EDGE_DIM), (0, 0)))
    b1 = p['in_mlp']['b1'].reshape(1, HIDDEN)
    w2 = p['in_mlp']['W2']
    b2 = p['in_mlp']['b2'].reshape(1, HIDDEN)

    units = list(p['blocks']) + [p['pma']]
    def stkw(name):
        return jnp.stack([u[name] for u in units])
    def stkb(name):
        return jnp.stack([u[name].reshape(1, HIDDEN) for u in units])
    WQ, WK, WV, WO = stkw('Wq'), stkw('Wk'), stkw('Wv'), stkw('Wo')
    F1W, F2W = stkw('ff1W'), stkw('ff2W')
    BQ, BK, BV, BO = stkb('bq'), stkb('bk'), stkb('bv'), stkb('bo')
    F1B, F2B = stkb('ff1b'), stkb('ff2b')
    L1G, L1B = stkb('ln1g'), stkb('ln1b')
    L2G, L2B = stkb('ln2g'), stkb('ln2b')
    seed = p['pma']['seed'].reshape(1, HIDDEN)
    ow1 = p['out_mlp']['W1']
    ob1 = p['out_mlp']['b1'].reshape(1, HIDDEN)
    ow2 = p['out_mlp']['W2'].reshape(1, HIDDEN)
    ob2 = jnp.broadcast_to(p['out_mlp']['b2'].reshape(1, 1), (1, 128))

    wspec = [
        pl.BlockSpec((NODE_DIM, HIDDEN), _c2),
        pl.BlockSpec((NODE_DIM, HIDDEN), _c2),
        pl.BlockSpec((128, HIDDEN), _c2),
        pl.BlockSpec((1, HIDDEN), _c2),
        pl.BlockSpec((HIDDEN, HIDDEN), _c2),
        pl.BlockSpec((1, HIDDEN), _c2),
    ]
    wspec += [pl.BlockSpec((4, HIDDEN, HIDDEN), _c3)] * 6
    wspec += [pl.BlockSpec((4, 1, HIDDEN), _c3)] * 10
    wspec += [
        pl.BlockSpec((1, HIDDEN), _c2),
        pl.BlockSpec((HIDDEN, HIDDEN), _c2),
        pl.BlockSpec((1, HIDDEN), _c2),
        pl.BlockSpec((1, HIDDEN), _c2),
        pl.BlockSpec((1, 128), _c2),
    ]

    out = pl.pallas_call(
        _body,
        grid=(N_GRAPHS,),
        in_specs=[
            pl.BlockSpec((1, NPG, NODE_DIM), _im3),
            pl.BlockSpec((1, EPG, 1), _im3),
            pl.BlockSpec((1, 1, EPG), _im3),
            pl.BlockSpec((1, EPG, 1), _im3),
            pl.BlockSpec((1, 1, EPG), _im3),
            pl.BlockSpec((1, EPG, 128), _im3),
        ] + wspec,
        out_specs=pl.BlockSpec((1, 1, 128), _im3),
        out_shape=jax.ShapeDtypeStruct((N_GRAPHS, 1, 128), jnp.float32),
    )(x3, sc, sr, dc, dr, ea,
      w1a, w1b, w1c, b1, w2, b2,
      WQ, WK, WV, WO, F1W, F2W,
      BQ, BK, BV, BO, F1B, F2B,
      L1G, L1B, L2G, L2B,
      seed, ow1, ob1, ow2, ob2)
    return out[:, 0, 0]


# speedup vs baseline: 1.6294x; 1.1045x over previous
import jax
import jax.numpy as jnp
import numpy as np
from jax.experimental import pallas as pl

NODE_DIM = 128
EDGE_DIM = 16
HIDDEN = 256
HEADS = 8
DH = HIDDEN // HEADS
N_GRAPHS = 64
NPG = 64
EPG = 256
NEG_INF = -1e9


def _ln(h, g, b):
    m = jnp.mean(h, axis=-1, keepdims=True)
    d = h - m
    v = jnp.mean(d * d, axis=-1, keepdims=True)
    return d * jax.lax.rsqrt(v + 1e-5) * g + b


def _mha(q_in, kv_in, wqkv, bqkv, wo, bo, sbias):
    scale = 1.0 / np.sqrt(DH)
    qkv = jnp.dot(kv_in, wqkv, preferred_element_type=jnp.float32) + bqkv
    if q_in is kv_in:
        q = qkv[:, :HIDDEN] * scale
    else:
        q = (jnp.dot(q_in, wqkv[:, :HIDDEN], preferred_element_type=jnp.float32)
             + bqkv[:, :HIDDEN]) * scale
    k = qkv[:, HIDDEN:2 * HIDDEN]
    v = qkv[:, 2 * HIDDEN:]
    outs = []
    for hh in range(HEADS):
        qh = q[:, hh * DH:(hh + 1) * DH]
        kh = k[:, hh * DH:(hh + 1) * DH]
        vh = v[:, hh * DH:(hh + 1) * DH]
        s = jax.lax.dot_general(qh, kh, (((1,), (1,)), ((), ())),
                                preferred_element_type=jnp.float32)
        if sbias is not None:
            s = s + sbias
        m = jnp.max(s, axis=-1, keepdims=True)
        e = jnp.exp(s - m)
        r = 1.0 / jnp.sum(e, axis=-1, keepdims=True)
        oh = jnp.dot(e, vh, preferred_element_type=jnp.float32)
        outs.append(oh * r)
    o = jnp.concatenate(outs, axis=-1)
    return jnp.dot(o, wo, preferred_element_type=jnp.float32) + bo


def _ff(h1, f1w, f1b, f2w, f2b):
    t = jnp.maximum(jnp.dot(h1, f1w, preferred_element_type=jnp.float32) + f1b, 0.0)
    return jnp.dot(t, f2w, preferred_element_type=jnp.float32) + f2b


def _body(xg_ref, sc_ref, sr_ref, dc_ref, dr_ref, ea_ref,
          w1a_ref, w1b_ref, w1c_ref, b1_ref, w2_ref, b2_ref,
          wqkv_ref, wo_ref, f1w_ref, f2w_ref,
          bqkv_ref, bo_ref, f1b_ref, f2b_ref,
          l1g_ref, l1b_ref, l2g_ref, l2b_ref,
          seed_ref, ow1_ref, ob1_ref, ow2_ref, ob2_ref,
          out_ref):
    xg = xg_ref[0]
    sc = sc_ref[0]
    sr = sr_ref[0]
    dc = dc_ref[0]
    dr = dr_ref[0]
    ea = ea_ref[0]

    adj = (sc == sr) | (sc == dr) | (dc == sr) | (dc == dr)
    sbias = jnp.where(adj, 0.0, NEG_INF)

    iota = jax.lax.broadcasted_iota(jnp.int32, (EPG, NPG), 1)
    ohs = (sc == iota).astype(jnp.float32)
    ohd = (dc == iota).astype(jnp.float32)
    a_proj = jnp.dot(xg, w1a_ref[...], preferred_element_type=jnp.float32)
    b_proj = jnp.dot(xg, w1b_ref[...], preferred_element_type=jnp.float32)
    pre = (jnp.dot(ohs, a_proj, preferred_element_type=jnp.float32)
           + jnp.dot(ohd, b_proj, preferred_element_type=jnp.float32)
           + jnp.dot(ea, w1c_ref[...], preferred_element_type=jnp.float32)
           + b1_ref[...])
    h = jnp.dot(jnp.maximum(pre, 0.0), w2_ref[...],
                preferred_element_type=jnp.float32) + b2_ref[...]

    for i in range(3):
        att = _mha(h, h, wqkv_ref[i], bqkv_ref[i], wo_ref[i], bo_ref[i],
                   sbias if i < 2 else None)
        h1 = _ln(h + att, l1g_ref[i], l1b_ref[i])
        f = _ff(h1, f1w_ref[i], f1b_ref[i], f2w_ref[i], f2b_ref[i])
        h = _ln(h1 + f, l2g_ref[i], l2b_ref[i])

    s_seed = seed_ref[...]
    att = _mha(s_seed, h, wqkv_ref[3], bqkv_ref[3], wo_ref[3], bo_ref[3], None)
    h1 = _ln(s_seed + att, l1g_ref[3], l1b_ref[3])
    f = _ff(h1, f1w_ref[3], f1b_ref[3], f2w_ref[3], f2b_ref[3])
    pooled = _ln(h1 + f, l2g_ref[3], l2b_ref[3])

    t = jnp.maximum(jnp.dot(pooled, ow1_ref[...],
                            preferred_element_type=jnp.float32) + ob1_ref[...], 0.0)
    logit = jnp.sum(t * ow2_ref[...], axis=-1, keepdims=True)
    out_ref[0] = jnp.broadcast_to(logit, (1, 128)) + ob2_ref[...]


def _im3(g):
    return (g, 0, 0)


def _c2(g):
    return (0, 0)


def _c3(g):
    return (0, 0, 0)


def kernel(x, edge_index, edge_attr, batch, params):
    src = edge_index[0].astype(jnp.int32)
    dst = edge_index[1].astype(jnp.int32)
    base = (jnp.arange(N_GRAPHS, dtype=jnp.int32) * NPG)[:, None]
    src_l = src.reshape(N_GRAPHS, EPG) - base
    dst_l = dst.reshape(N_GRAPHS, EPG) - base
    sc = src_l.reshape(N_GRAPHS, EPG, 1)
    sr = src_l.reshape(N_GRAPHS, 1, EPG)
    dc = dst_l.reshape(N_GRAPHS, EPG, 1)
    dr = dst_l.reshape(N_GRAPHS, 1, EPG)
    x3 = x.reshape(N_GRAPHS, NPG, NODE_DIM)
    ea = jnp.pad(edge_attr, ((0, 0), (0, 128 - EDGE_DIM))).reshape(N_GRAPHS, EPG, 128)

    p = params
    w1 = p['in_mlp']['W1']
    w1a = w1[:NODE_DIM]
    w1b = w1[NODE_DIM:2 * NODE_DIM]
    w1c = jnp.pad(w1[2 * NODE_DIM:], ((0, 128 - EDGE_DIM), (0, 0)))
    b1 = p['in_mlp']['b1'].reshape(1, HIDDEN)
    w2 = p['in_mlp']['W2']
    b2 = p['in_mlp']['b2'].reshape(1, HIDDEN)

    units = list(p['blocks']) + [p['pma']]
    def stkw(name):
        return jnp.stack([u[name] for u in units])
    def stkb(name):
        return jnp.stack([u[name].reshape(1, HIDDEN) for u in units])
    WQKV = jnp.stack([jnp.concatenate([u['Wq'], u['Wk'], u['Wv']], axis=1)
                      for u in units])
    BQKV = jnp.stack([jnp.concatenate([u['bq'], u['bk'], u['bv']]).reshape(1, 3 * HIDDEN)
                      for u in units])
    WO = stkw('Wo')
    F1W, F2W = stkw('ff1W'), stkw('ff2W')
    BO = stkb('bo')
    F1B, F2B = stkb('ff1b'), stkb('ff2b')
    L1G, L1B = stkb('ln1g'), stkb('ln1b')
    L2G, L2B = stkb('ln2g'), stkb('ln2b')
    seed = p['pma']['seed'].reshape(1, HIDDEN)
    ow1 = p['out_mlp']['W1']
    ob1 = p['out_mlp']['b1'].reshape(1, HIDDEN)
    ow2 = p['out_mlp']['W2'].reshape(1, HIDDEN)
    ob2 = jnp.broadcast_to(p['out_mlp']['b2'].reshape(1, 1), (1, 128))

    wspec = [
        pl.BlockSpec((NODE_DIM, HIDDEN), _c2),
        pl.BlockSpec((NODE_DIM, HIDDEN), _c2),
        pl.BlockSpec((128, HIDDEN), _c2),
        pl.BlockSpec((1, HIDDEN), _c2),
        pl.BlockSpec((HIDDEN, HIDDEN), _c2),
        pl.BlockSpec((1, HIDDEN), _c2),
        pl.BlockSpec((4, HIDDEN, 3 * HIDDEN), _c3),
        pl.BlockSpec((4, HIDDEN, HIDDEN), _c3),
        pl.BlockSpec((4, HIDDEN, HIDDEN), _c3),
        pl.BlockSpec((4, HIDDEN, HIDDEN), _c3),
        pl.BlockSpec((4, 1, 3 * HIDDEN), _c3),
    ]
    wspec += [pl.BlockSpec((4, 1, HIDDEN), _c3)] * 7
    wspec += [
        pl.BlockSpec((1, HIDDEN), _c2),
        pl.BlockSpec((HIDDEN, HIDDEN), _c2),
        pl.BlockSpec((1, HIDDEN), _c2),
        pl.BlockSpec((1, HIDDEN), _c2),
        pl.BlockSpec((1, 128), _c2),
    ]

    out = pl.pallas_call(
        _body,
        grid=(N_GRAPHS,),
        in_specs=[
            pl.BlockSpec((1, NPG, NODE_DIM), _im3),
            pl.BlockSpec((1, EPG, 1), _im3),
            pl.BlockSpec((1, 1, EPG), _im3),
            pl.BlockSpec((1, EPG, 1), _im3),
            pl.BlockSpec((1, 1, EPG), _im3),
            pl.BlockSpec((1, EPG, 128), _im3),
        ] + wspec,
        out_specs=pl.BlockSpec((1, 1, 128), _im3),
        out_shape=jax.ShapeDtypeStruct((N_GRAPHS, 1, 128), jnp.float32),
    )(x3, sc, sr, dc, dr, ea,
      w1a, w1b, w1c, b1, w2, b2,
      WQKV, WO, F1W, F2W,
      BQKV, BO, F1B, F2B,
      L1G, L1B, L2G, L2B,
      seed, ow1, ob1, ow2, ob2)
    return out[:, 0, 0]
